# baseline (speedup 1.0000x reference)
import functools

import jax
import jax.numpy as jnp
from jax import lax
from jax.experimental import pallas as pl
from jax.experimental.pallas import tpu as pltpu
from jax.experimental.pallas import tpu_sc as plsc

NC = 2
NS = 16
NW = NC * NS
DG = 16


def _make_seg_sum(n, e, d, ch, nbuf, with_deg):
  et = e // NW
  nchunks = et // ch
  etail = et % ch
  nslot = 2 * nbuf
  ngroups = nchunks // nslot
  ntail = nchunks % nslot
  assert ch % 8 == 0 and (nchunks * ch) % 8 == 0 and nchunks >= nslot
  rpt = (n // NS + 7) // 8 * 8
  npad = rpt * NS

  out_type = [jax.ShapeDtypeStruct((NC, npad, d), jnp.float32)]
  scratch = [
      pltpu.VMEM((nslot, 2, ch), jnp.int32),
      pltpu.VMEM((nbuf, ch, d), jnp.float32),
      pltpu.VMEM_SHARED((npad, d), jnp.float32),
      pltpu.SemaphoreType.DMA((nslot,)),
      pltpu.SemaphoreType.DMA((nslot,)),
      pltpu.SemaphoreType.DMA((nbuf,)),
      pltpu.SemaphoreType.DMA((nbuf,)),
  ]
  if with_deg:
    out_type.append(jax.ShapeDtypeStruct((NC, npad, DG), jnp.float32))
    scratch.append(pltpu.VMEM((ch, DG), jnp.float32))
    scratch.append(pltpu.VMEM_SHARED((npad, DG), jnp.float32))
  if etail:
    scratch.append(pltpu.VMEM((2, etail), jnp.int32))
    scratch.append(pltpu.VMEM((etail, d), jnp.float32))
    scratch.append(pltpu.SemaphoreType.DMA)

  mesh = plsc.VectorSubcoreMesh(core_axis_name="c", subcore_axis_name="s")

  def body(tab_hbm, ei_hbm, zrows_hbm, zdeg_hbm, out_hbm, deg_hbm,
           idx_v, rows_v, table_s, isems_s, isems_d, gsems, ssems,
           ones_v, degtab_s, tidx_v, trows_v, tsem):
    c = lax.axis_index("c")
    s = lax.axis_index("s")
    wid = s * NC + c
    ebase = wid * et

    pltpu.sync_copy(zrows_hbm, table_s.at[pl.ds(s * rpt, rpt)])
    if with_deg:
      pltpu.sync_copy(zdeg_hbm, degtab_s.at[pl.ds(s * rpt, rpt)])
      for i in range(ch):
        ones_v[i] = jnp.ones((DG,), jnp.float32)

    def load_idx(j, q):
      eb = ebase + j * ch
      pltpu.async_copy(ei_hbm.at[0, pl.ds(eb, ch)], idx_v.at[q, 0],
                       isems_s.at[q])
      pltpu.async_copy(ei_hbm.at[1, pl.ds(eb, ch)], idx_v.at[q, 1],
                       isems_d.at[q])

    def issue_gather(q, b):
      pltpu.make_async_copy(ei_hbm.at[0, pl.ds(0, ch)], idx_v.at[q, 0],
                            isems_s.at[q]).wait()
      pltpu.async_copy(tab_hbm.at[idx_v.at[q, 0]], rows_v.at[b], gsems.at[b])

    for q in range(nslot):
      load_idx(q, q)
    for b in range(nbuf):
      issue_gather(b, b)

    plsc.subcore_barrier()

    if etail:
      tb = ebase + nchunks * ch
      pltpu.sync_copy(ei_hbm.at[0, pl.ds(tb, etail)], tidx_v.at[0])
      pltpu.sync_copy(ei_hbm.at[1, pl.ds(tb, etail)], tidx_v.at[1])
      pltpu.async_copy(tab_hbm.at[tidx_v.at[0]], trows_v, tsem).wait()
      pltpu.sync_copy(trows_v, table_s.at[tidx_v.at[1]], add=True)
      if with_deg:
        pltpu.sync_copy(ones_v.at[pl.ds(0, etail)],
                        degtab_s.at[tidx_v.at[1]], add=True)

    def drain_scatter(bp):
      pltpu.make_async_copy(tab_hbm.at[pl.ds(0, ch)], rows_v.at[bp],
                            ssems.at[bp]).wait()

    def stage(j, u, guard):
      b = u % nbuf
      if with_deg:
        pltpu.make_async_copy(tab_hbm.at[pl.ds(0, ch)], rows_v.at[b],
                              gsems.at[b]).wait()
        pltpu.make_async_copy(ei_hbm.at[0, pl.ds(0, ch)], idx_v.at[u, 1],
                              isems_d.at[u]).wait()
        pltpu.sync_copy(rows_v.at[b], table_s.at[idx_v.at[u, 1]], add=True)
        pltpu.sync_copy(ones_v, degtab_s.at[idx_v.at[u, 1]], add=True)
        guard(j + nslot < nchunks, lambda: load_idx(j + nslot, u))
        guard(j + nbuf < nchunks,
              lambda: issue_gather((u + nbuf) % nslot, b))
      else:
        up = (u - 1) % nslot
        bp = up % nbuf

        def lag():
          drain_scatter(bp)
          guard(j - 1 + nslot < nchunks, lambda: load_idx(j - 1 + nslot, up))
          guard(j - 1 + nbuf < nchunks,
                lambda: issue_gather((up + nbuf) % nslot, bp))

        guard(j >= 1, lag)
        pltpu.make_async_copy(tab_hbm.at[pl.ds(0, ch)], rows_v.at[b],
                              gsems.at[b]).wait()
        pltpu.make_async_copy(ei_hbm.at[0, pl.ds(0, ch)], idx_v.at[u, 1],
                              isems_d.at[u]).wait()
        pltpu.async_copy(rows_v.at[b], table_s.at[idx_v.at[u, 1]],
                         ssems.at[b], add=True)

    def traced_guard(cond, fn):
      pl.when(cond)(fn)

    def static_guard(cond, fn):
      if cond:
        fn()

    def group(g, carry):
      for u in range(nslot):
        stage(g * nslot + u, u, traced_guard)
      return carry

    lax.fori_loop(0, ngroups, group, 0, unroll=False)
    for u in range(ntail):
      stage(ngroups * nslot + u, u, static_guard)
    if not with_deg:
      drain_scatter((nchunks - 1) % nbuf)
    plsc.subcore_barrier()

    pltpu.sync_copy(table_s.at[pl.ds(s * rpt, rpt)],
                    out_hbm.at[c, pl.ds(s * rpt, rpt)])
    if with_deg:
      pltpu.sync_copy(degtab_s.at[pl.ds(s * rpt, rpt)],
                      deg_hbm.at[c, pl.ds(s * rpt, rpt)])

  kern = functools.partial(
      pl.kernel,
      mesh=mesh,
      compiler_params=pltpu.CompilerParams(use_tc_tiling_on_sc=False),
      out_type=tuple(out_type) if with_deg else out_type[0],
      scratch_types=scratch,
  )

  @kern
  def seg(*refs):
    it = iter(refs)
    tab, ei, zrows = next(it), next(it), next(it)
    zdeg = next(it) if with_deg else None
    out = next(it)
    deg = next(it) if with_deg else None
    idx_v, rows_v, table_s = next(it), next(it), next(it)
    isems_s, isems_d, gsems, ssems = next(it), next(it), next(it), next(it)
    ones_v = next(it) if with_deg else None
    degtab_s = next(it) if with_deg else None
    tidx_v = next(it) if etail else None
    trows_v = next(it) if etail else None
    tsem = next(it) if etail else None
    body(tab, ei, zrows, zdeg, out, deg,
         idx_v, rows_v, table_s, isems_s, isems_d, gsems, ssems,
         ones_v, degtab_s, tidx_v, trows_v, tsem)

  return seg


def _layer1_body(feat_ref, degt_ref, x_ref, w1_ref, b1_ref, w2_ref,
                 p_ref, invd_ref):
  aggx = feat_ref[0] + feat_ref[1] + x_ref[...]
  deg16 = degt_ref[0] + degt_ref[1]
  inv = 1.0 / (jnp.max(deg16, axis=1, keepdims=True) + 1.0)
  h = aggx * inv
  h1 = jnp.maximum(
      jnp.dot(h, w1_ref[...], preferred_element_type=jnp.float32)
      + b1_ref[...], 0.0)
  p_ref[...] = jnp.dot(h1, w2_ref[...], preferred_element_type=jnp.float32)
  invd_ref[...] = jnp.broadcast_to(inv, invd_ref.shape)


def _layer2_body(agg_ref, p_ref, invd_ref, b2_ref, z_ref):
  z_ref[...] = ((agg_ref[0] + agg_ref[1] + p_ref[...]) * invd_ref[...]
                + b2_ref[...])


def _decoder_body(zr_ref, zc_ref, out_ref):
  out_ref[...] = lax.dot_general(
      zr_ref[...], zc_ref[...], (((1,), (1,)), ((), ())),
      preferred_element_type=jnp.float32)


def kernel(x, edge_index, W1, b1, W2, b2):
  n, din = x.shape
  e = edge_index.shape[1]
  h1_dim = W1.shape[1]
  h2 = W2.shape[1]

  rpt = (n // NS + 7) // 8 * 8
  z1 = jnp.zeros((rpt, din), jnp.float32)
  zd = jnp.zeros((rpt, DG), jnp.float32)
  z2 = jnp.zeros((rpt, h2), jnp.float32)

  seg1 = _make_seg_sum(n, e, din, 80, 3, with_deg=True)
  seg2 = _make_seg_sum(n, e, h2, 128, 3, with_deg=False)

  feat, degt = seg1(x, edge_index, z1, zd)

  rblk = 2000
  grid1 = n // rblk
  p, invd = pl.pallas_call(
      _layer1_body,
      grid=(grid1,),
      in_specs=[
          pl.BlockSpec((NC, rblk, din), lambda i: (0, i, 0)),
          pl.BlockSpec((NC, rblk, DG), lambda i: (0, i, 0)),
          pl.BlockSpec((rblk, din), lambda i: (i, 0)),
          pl.BlockSpec((din, h1_dim), lambda i: (0, 0)),
          pl.BlockSpec((1, h1_dim), lambda i: (0, 0)),
          pl.BlockSpec((h1_dim, h2), lambda i: (0, 0)),
      ],
      out_specs=[
          pl.BlockSpec((rblk, h2), lambda i: (i, 0)),
          pl.BlockSpec((rblk, h2), lambda i: (i, 0)),
      ],
      out_shape=[
          jax.ShapeDtypeStruct((n, h2), jnp.float32),
          jax.ShapeDtypeStruct((n, h2), jnp.float32),
      ],
  )(feat, degt, x, W1, b1.reshape(1, h1_dim), W2)

  agg2 = seg2(p, edge_index, z2)

  z = pl.pallas_call(
      _layer2_body,
      grid=(grid1,),
      in_specs=[
          pl.BlockSpec((NC, rblk, h2), lambda i: (0, i, 0)),
          pl.BlockSpec((rblk, h2), lambda i: (i, 0)),
          pl.BlockSpec((rblk, h2), lambda i: (i, 0)),
          pl.BlockSpec((1, h2), lambda i: (0, 0)),
      ],
      out_specs=pl.BlockSpec((rblk, h2), lambda i: (i, 0)),
      out_shape=jax.ShapeDtypeStruct((n, h2), jnp.float32),
  )(agg2, p, invd, b2.reshape(1, h2))

  ablk = 512
  grid_a = pl.cdiv(n, ablk)
  adj = pl.pallas_call(
      _decoder_body,
      grid=(grid_a,),
      in_specs=[
          pl.BlockSpec((ablk, h2), lambda i: (i, 0)),
          pl.BlockSpec((n, h2), lambda i: (0, 0)),
      ],
      out_specs=pl.BlockSpec((ablk, n), lambda i: (i, 0)),
      out_shape=jax.ShapeDtypeStruct((n, n), jnp.float32),
      compiler_params=pltpu.CompilerParams(
          vmem_limit_bytes=100 * 1024 * 1024),
  )(z, z)

  return (z, adj)


# speedup vs baseline: 1.0026x; 1.0026x over previous
import functools

import jax
import jax.numpy as jnp
from jax import lax
from jax.experimental import pallas as pl
from jax.experimental.pallas import tpu as pltpu
from jax.experimental.pallas import tpu_sc as plsc

NC = 2
NS = 16
NW = NC * NS
DG = 16


def _make_seg_sum(n, e, d, ch, nbuf, with_deg):
  et = e // NW
  nchunks = et // ch
  etail = et % ch
  nslot = 2 * nbuf
  ngroups = nchunks // nslot
  ntail = nchunks % nslot
  assert ch % 8 == 0 and (nchunks * ch) % 8 == 0 and nchunks >= nslot
  rpt = (n // NS + 7) // 8 * 8
  npad = rpt * NS

  out_type = [jax.ShapeDtypeStruct((NC, npad, d), jnp.float32)]
  scratch = [
      pltpu.VMEM((nslot, 2, ch), jnp.int32),
      pltpu.VMEM((nbuf, ch, d), jnp.float32),
      pltpu.VMEM_SHARED((npad, d), jnp.float32),
      pltpu.SemaphoreType.DMA((nslot,)),
      pltpu.SemaphoreType.DMA((nbuf,)),
  ]
  if with_deg:
    out_type.append(jax.ShapeDtypeStruct((NC, npad, DG), jnp.float32))
    scratch.append(pltpu.VMEM((ch, DG), jnp.float32))
    scratch.append(pltpu.VMEM_SHARED((npad, DG), jnp.float32))
  if etail:
    scratch.append(pltpu.VMEM((2, etail), jnp.int32))
    scratch.append(pltpu.VMEM((etail, d), jnp.float32))
    scratch.append(pltpu.SemaphoreType.DMA)

  mesh = plsc.VectorSubcoreMesh(core_axis_name="c", subcore_axis_name="s")

  def body(tab_hbm, ei_hbm, zrows_hbm, zdeg_hbm, out_hbm, deg_hbm,
           idx_v, rows_v, table_s, isems, gsems,
           ones_v, degtab_s, tidx_v, trows_v, tsem):
    c = lax.axis_index("c")
    s = lax.axis_index("s")
    wid = s * NC + c
    ebase = wid * et

    pltpu.sync_copy(zrows_hbm, table_s.at[pl.ds(s * rpt, rpt)])
    if with_deg:
      pltpu.sync_copy(zdeg_hbm, degtab_s.at[pl.ds(s * rpt, rpt)])
      for i in range(ch):
        ones_v[i] = jnp.ones((DG,), jnp.float32)

    def load_idx(j, q):
      eb = ebase + j * ch
      pltpu.async_copy(ei_hbm.at[pl.ds(0, 2), pl.ds(eb, ch)], idx_v.at[q],
                       isems.at[q])

    def issue_gather(q, b):
      pltpu.make_async_copy(ei_hbm.at[pl.ds(0, 2), pl.ds(0, ch)], idx_v.at[q],
                            isems.at[q]).wait()
      pltpu.async_copy(tab_hbm.at[idx_v.at[q, 0]], rows_v.at[b], gsems.at[b])

    for q in range(nslot):
      load_idx(q, q)
    for b in range(nbuf):
      issue_gather(b, b)

    plsc.subcore_barrier()

    if etail:
      tb = ebase + nchunks * ch
      pltpu.sync_copy(ei_hbm.at[pl.ds(0, 2), pl.ds(tb, etail)], tidx_v)
      pltpu.async_copy(tab_hbm.at[tidx_v.at[0]], trows_v, tsem).wait()
      pltpu.sync_copy(trows_v, table_s.at[tidx_v.at[1]], add=True)
      if with_deg:
        pltpu.sync_copy(ones_v.at[pl.ds(0, etail)],
                        degtab_s.at[tidx_v.at[1]], add=True)

    def stage(j, u, guard):
      b = u % nbuf
      pltpu.make_async_copy(tab_hbm.at[pl.ds(0, ch)], rows_v.at[b],
                            gsems.at[b]).wait()
      pltpu.sync_copy(rows_v.at[b], table_s.at[idx_v.at[u, 1]], add=True)
      if with_deg:
        pltpu.sync_copy(ones_v, degtab_s.at[idx_v.at[u, 1]], add=True)
      guard(j + nslot < nchunks, lambda: load_idx(j + nslot, u))
      guard(j + nbuf < nchunks,
            lambda: issue_gather((u + nbuf) % nslot, b))

    def traced_guard(cond, fn):
      pl.when(cond)(fn)

    def static_guard(cond, fn):
      if cond:
        fn()

    def group(g, carry):
      for u in range(nslot):
        stage(g * nslot + u, u, traced_guard)
      return carry

    lax.fori_loop(0, ngroups, group, 0, unroll=False)
    for u in range(ntail):
      stage(ngroups * nslot + u, u, static_guard)
    plsc.subcore_barrier()

    pltpu.sync_copy(table_s.at[pl.ds(s * rpt, rpt)],
                    out_hbm.at[c, pl.ds(s * rpt, rpt)])
    if with_deg:
      pltpu.sync_copy(degtab_s.at[pl.ds(s * rpt, rpt)],
                      deg_hbm.at[c, pl.ds(s * rpt, rpt)])

  kern = functools.partial(
      pl.kernel,
      mesh=mesh,
      compiler_params=pltpu.CompilerParams(use_tc_tiling_on_sc=False),
      out_type=tuple(out_type) if with_deg else out_type[0],
      scratch_types=scratch,
  )

  @kern
  def seg(*refs):
    it = iter(refs)
    tab, ei, zrows = next(it), next(it), next(it)
    zdeg = next(it) if with_deg else None
    out = next(it)
    deg = next(it) if with_deg else None
    idx_v, rows_v, table_s = next(it), next(it), next(it)
    isems, gsems = next(it), next(it)
    ones_v = next(it) if with_deg else None
    degtab_s = next(it) if with_deg else None
    tidx_v = next(it) if etail else None
    trows_v = next(it) if etail else None
    tsem = next(it) if etail else None
    body(tab, ei, zrows, zdeg, out, deg,
         idx_v, rows_v, table_s, isems, gsems,
         ones_v, degtab_s, tidx_v, trows_v, tsem)

  return seg


def _layer1_body(feat_ref, degt_ref, x_ref, w1_ref, b1_ref, w2_ref,
                 p_ref, invd_ref):
  aggx = feat_ref[0] + feat_ref[1] + x_ref[...]
  deg16 = degt_ref[0] + degt_ref[1]
  inv = 1.0 / (jnp.max(deg16, axis=1, keepdims=True) + 1.0)
  h = aggx * inv
  h1 = jnp.maximum(
      jnp.dot(h, w1_ref[...], preferred_element_type=jnp.float32)
      + b1_ref[...], 0.0)
  p_ref[...] = jnp.dot(h1, w2_ref[...], preferred_element_type=jnp.float32)
  invd_ref[...] = jnp.broadcast_to(inv, invd_ref.shape)


def _layer2_body(agg_ref, p_ref, invd_ref, b2_ref, z_ref):
  z_ref[...] = ((agg_ref[0] + agg_ref[1] + p_ref[...]) * invd_ref[...]
                + b2_ref[...])


def _decoder_body(zr_ref, zc_ref, out_ref):
  out_ref[...] = lax.dot_general(
      zr_ref[...], zc_ref[...], (((1,), (1,)), ((), ())),
      preferred_element_type=jnp.float32)


def kernel(x, edge_index, W1, b1, W2, b2):
  n, din = x.shape
  e = edge_index.shape[1]
  h1_dim = W1.shape[1]
  h2 = W2.shape[1]

  rpt = (n // NS + 7) // 8 * 8
  z1 = jnp.zeros((rpt, din), jnp.float32)
  zd = jnp.zeros((rpt, DG), jnp.float32)
  z2 = jnp.zeros((rpt, h2), jnp.float32)

  seg1 = _make_seg_sum(n, e, din, 80, 3, with_deg=True)
  seg2 = _make_seg_sum(n, e, h2, 128, 3, with_deg=False)

  feat, degt = seg1(x, edge_index, z1, zd)

  rblk = 2000
  grid1 = n // rblk
  p, invd = pl.pallas_call(
      _layer1_body,
      grid=(grid1,),
      in_specs=[
          pl.BlockSpec((NC, rblk, din), lambda i: (0, i, 0)),
          pl.BlockSpec((NC, rblk, DG), lambda i: (0, i, 0)),
          pl.BlockSpec((rblk, din), lambda i: (i, 0)),
          pl.BlockSpec((din, h1_dim), lambda i: (0, 0)),
          pl.BlockSpec((1, h1_dim), lambda i: (0, 0)),
          pl.BlockSpec((h1_dim, h2), lambda i: (0, 0)),
      ],
      out_specs=[
          pl.BlockSpec((rblk, h2), lambda i: (i, 0)),
          pl.BlockSpec((rblk, h2), lambda i: (i, 0)),
      ],
      out_shape=[
          jax.ShapeDtypeStruct((n, h2), jnp.float32),
          jax.ShapeDtypeStruct((n, h2), jnp.float32),
      ],
  )(feat, degt, x, W1, b1.reshape(1, h1_dim), W2)

  agg2 = seg2(p, edge_index, z2)

  z = pl.pallas_call(
      _layer2_body,
      grid=(grid1,),
      in_specs=[
          pl.BlockSpec((NC, rblk, h2), lambda i: (0, i, 0)),
          pl.BlockSpec((rblk, h2), lambda i: (i, 0)),
          pl.BlockSpec((rblk, h2), lambda i: (i, 0)),
          pl.BlockSpec((1, h2), lambda i: (0, 0)),
      ],
      out_specs=pl.BlockSpec((rblk, h2), lambda i: (i, 0)),
      out_shape=jax.ShapeDtypeStruct((n, h2), jnp.float32),
  )(agg2, p, invd, b2.reshape(1, h2))

  ablk = 512
  grid_a = pl.cdiv(n, ablk)
  adj = pl.pallas_call(
      _decoder_body,
      grid=(grid_a,),
      in_specs=[
          pl.BlockSpec((ablk, h2), lambda i: (i, 0)),
          pl.BlockSpec((n, h2), lambda i: (0, 0)),
      ],
      out_specs=pl.BlockSpec((ablk, n), lambda i: (i, 0)),
      out_shape=jax.ShapeDtypeStruct((n, n), jnp.float32),
      compiler_params=pltpu.CompilerParams(
          vmem_limit_bytes=100 * 1024 * 1024),
  )(z, z)

  return (z, adj)
